# baseline (speedup 1.0000x reference)
import jax
import jax.numpy as jnp
from jax import lax
from jax.experimental import pallas as pl
from jax.experimental.pallas import tpu as pltpu
from jax.experimental.pallas import tpu_sc as plsc

N = 10000
E = 320000
D_IN = 128
D_OUT = 128
NUM_CORES = 2
NUM_TILES = 16
NUM_WORKERS = NUM_CORES * NUM_TILES
EDGES_PER_WORKER = E // NUM_WORKERS
CHUNK = 64
CHUNKS_PER_WORKER = 162
EDGES_PAD = CHUNKS_PER_WORKER * CHUNK
IDX_BITS = 14
ROWS_PER_TILE = (N // NUM_TILES) // 8 * 8
ROWS_REM = N - NUM_TILES * ROWS_PER_TILE
BN = 1000


def _tc_support_body(x_ref, w_ref, t_ref, out_ref):
    s = jnp.dot(x_ref[...], w_ref[...], preferred_element_type=jnp.float32)
    out_ref[...] = s * t_ref[...]


def _tc_support(x, W, t2):
    return pl.pallas_call(
        _tc_support_body,
        grid=(N // BN,),
        in_specs=[
            pl.BlockSpec((BN, D_IN), lambda i: (i, 0)),
            pl.BlockSpec((D_IN, D_OUT), lambda i: (0, 0)),
            pl.BlockSpec((BN, 1), lambda i: (i, 0)),
        ],
        out_specs=pl.BlockSpec((BN, D_OUT), lambda i: (i, 0)),
        out_shape=jax.ShapeDtypeStruct((N, D_OUT), jnp.float32),
    )(x, W, t2)


def _tc_combine_body(a_ref, b_ref, out_ref):
    out_ref[...] = a_ref[0] + b_ref[0]


def _tc_combine(halves):
    return pl.pallas_call(
        _tc_combine_body,
        grid=(N // BN,),
        in_specs=[
            pl.BlockSpec((1, BN, D_OUT), lambda i: (0, i, 0)),
            pl.BlockSpec((1, BN, D_OUT), lambda i: (1, i, 0)),
        ],
        out_specs=pl.BlockSpec((BN, D_OUT), lambda i: (i, 0)),
        out_shape=jax.ShapeDtypeStruct((N, D_OUT), jnp.float32),
    )(halves, halves)


def _scale_chunk(gbuf_s, w_v, k):
    def group_body(g, carry2):
        wv = w_v[pl.ds(k * CHUNK + g * 16, 16)]
        for i in range(16):
            e = g * 16 + i
            ii = jnp.full((16,), i, jnp.int32)
            wb = lax.gather(
                wv, ii[:, None],
                lax.GatherDimensionNumbers(
                    offset_dims=(), collapsed_slice_dims=(0,),
                    start_index_map=(0,)),
                (1,),
                mode=lax.GatherScatterMode.PROMISE_IN_BOUNDS)
            for q in range(D_OUT // 16):
                gbuf_s[e, pl.ds(q * 16, 16)] = gbuf_s[e, pl.ds(q * 16, 16)] * wb
        return carry2

    lax.fori_loop(0, CHUNK // 16, group_body, 0)


def _sc_body(sup_ref, packed_ref, w_ref, binit_ref, out_ref,
             acc, packed_v, w_v, gbuf, srcidx, dstidx,
             gsem0, gsem1, gsem2, ssem0, ssem1, ssem2):
    c = lax.axis_index("c")
    tid = lax.axis_index("s")
    gsem = (gsem0, gsem1, gsem2)
    ssem = (ssem0, ssem1, ssem2)

    pltpu.sync_copy(binit_ref.at[c], acc.at[pl.ds(tid * ROWS_PER_TILE, ROWS_PER_TILE)])

    @pl.when(tid == NUM_TILES - 1)
    def _init_rem():
        pltpu.sync_copy(binit_ref.at[c, pl.ds(0, ROWS_REM)],
                        acc.at[pl.ds(NUM_TILES * ROWS_PER_TILE, ROWS_REM)])

    plsc.subcore_barrier()

    pltpu.sync_copy(packed_ref.at[c, tid], packed_v)
    pltpu.sync_copy(w_ref.at[c, tid], w_v)

    def unpack(k, s):
        for g in range(CHUNK // 16):
            p = packed_v[pl.ds(k * CHUNK + g * 16, 16)]
            srcidx[s, pl.ds(g * 16, 16)] = p & ((1 << IDX_BITS) - 1)
            dstidx[s, pl.ds(g * 16, 16)] = lax.shift_right_logical(
                p, jnp.full((16,), IDX_BITS, jnp.int32))

    def start_gather(s):
        pltpu.async_copy(sup_ref.at[srcidx.at[s]], gbuf.at[s], gsem[s])

    def wait_gather(s):
        pltpu.make_async_copy(sup_ref.at[srcidx.at[s]], gbuf.at[s], gsem[s]).wait()

    def start_scatter(s):
        pltpu.async_copy(gbuf.at[s], acc.at[dstidx.at[s]], ssem[s], add=True)

    def wait_scatter(s):
        pltpu.make_async_copy(gbuf.at[s], acc.at[dstidx.at[s]], ssem[s]).wait()

    def step(k, s, wait_sc, issue):
        wait_gather(s)
        _scale_chunk(gbuf.at[s], w_v, k)
        start_scatter(s)
        if wait_sc:
            wait_scatter((s + 2) % 3)
        if issue:
            unpack(k + 2, (s + 2) % 3)
            start_gather((s + 2) % 3)

    unpack(0, 0)
    start_gather(0)
    unpack(1, 1)
    start_gather(1)

    step(0, 0, False, True)
    step(1, 1, True, True)
    step(2, 2, True, True)

    def period(m, carry):
        k0 = 3 * m
        step(k0, 0, True, True)
        step(k0 + 1, 1, True, True)
        step(k0 + 2, 2, True, True)
        return carry

    lax.fori_loop(1, CHUNKS_PER_WORKER // 3 - 1, period, 0)

    kl = CHUNKS_PER_WORKER - 3
    step(kl, 0, True, True)
    step(kl + 1, 1, True, False)
    step(kl + 2, 2, True, False)
    wait_scatter(2)

    plsc.subcore_barrier()

    r0 = tid * ROWS_PER_TILE
    pltpu.sync_copy(
        acc.at[pl.ds(r0, ROWS_PER_TILE), :],
        out_ref.at[c, pl.ds(r0, ROWS_PER_TILE), :])

    @pl.when(tid == NUM_TILES - 1)
    def _out_rem():
        rr = NUM_TILES * ROWS_PER_TILE
        pltpu.sync_copy(acc.at[pl.ds(rr, ROWS_REM), :],
                        out_ref.at[c, pl.ds(rr, ROWS_REM), :])


def _sc_spmm(support, packedr, wr, binit):
    mesh = plsc.VectorSubcoreMesh(core_axis_name="c", subcore_axis_name="s")
    kern = pl.kernel(
        _sc_body,
        mesh=mesh,
        out_type=jax.ShapeDtypeStruct((2, N, D_OUT), jnp.float32),
        scratch_types=[
            pltpu.VMEM_SHARED((N, D_OUT), jnp.float32),
            pltpu.VMEM((EDGES_PAD,), jnp.int32),
            pltpu.VMEM((EDGES_PAD,), jnp.float32),
            pltpu.VMEM((3, CHUNK, D_OUT), jnp.float32),
            pltpu.VMEM((3, CHUNK), jnp.int32),
            pltpu.VMEM((3, CHUNK), jnp.int32),
            pltpu.SemaphoreType.DMA,
            pltpu.SemaphoreType.DMA,
            pltpu.SemaphoreType.DMA,
            pltpu.SemaphoreType.DMA,
            pltpu.SemaphoreType.DMA,
            pltpu.SemaphoreType.DMA,
        ],
    )
    return kern(support, packedr, wr, binit)


def kernel(input, edge_index, edge_weight, t, W, b):
    x = input.astype(jnp.float32)
    t2 = t.reshape(N, 1)
    support = _tc_support(x, W, t2)

    src = edge_index[0].astype(jnp.int32)
    dst = edge_index[1].astype(jnp.int32)
    packed = ((dst << IDX_BITS) | src).reshape(NUM_WORKERS, EDGES_PER_WORKER)
    w = edge_weight.reshape(NUM_WORKERS, EDGES_PER_WORKER)
    pad = EDGES_PAD - EDGES_PER_WORKER
    eshape = (NUM_CORES, NUM_TILES, EDGES_PAD)
    packedr = jnp.pad(packed, ((0, 0), (0, pad))).reshape(eshape)
    wr = jnp.pad(w, ((0, 0), (0, pad))).reshape(eshape)

    binit = jnp.stack([
        jnp.broadcast_to(b.reshape(1, D_OUT), (ROWS_PER_TILE, D_OUT)),
        jnp.zeros((ROWS_PER_TILE, D_OUT), jnp.float32),
    ])

    halves = _sc_spmm(support, packedr, wr, binit)
    return _tc_combine(halves)


# speedup vs baseline: 1.5277x; 1.5277x over previous
import jax
import jax.numpy as jnp
from jax import lax
from jax.experimental import pallas as pl
from jax.experimental.pallas import tpu as pltpu
from jax.experimental.pallas import tpu_sc as plsc

N = 10000
E = 320000
D_IN = 128
D_OUT = 128
NUM_CORES = 2
NUM_TILES = 16
NUM_WORKERS = NUM_CORES * NUM_TILES
EDGES_PER_WORKER = E // NUM_WORKERS
CHUNK = 128
CHUNKS_PER_WORKER = 79
EDGES_PAD = CHUNKS_PER_WORKER * CHUNK
ROWS_PER_TILE = (N // NUM_TILES) // 8 * 8
ROWS_REM = N - NUM_TILES * ROWS_PER_TILE
BN = 1000


def _tc_support_body(x_ref, w_ref, t_ref, out_ref):
    s = jnp.dot(x_ref[...], w_ref[...], preferred_element_type=jnp.float32)
    out_ref[...] = s * t_ref[...]


def _tc_support(x, W, t2):
    return pl.pallas_call(
        _tc_support_body,
        grid=(N // BN,),
        in_specs=[
            pl.BlockSpec((BN, D_IN), lambda i: (i, 0)),
            pl.BlockSpec((D_IN, D_OUT), lambda i: (0, 0)),
            pl.BlockSpec((BN, 1), lambda i: (i, 0)),
        ],
        out_specs=pl.BlockSpec((BN, D_OUT), lambda i: (i, 0)),
        out_shape=jax.ShapeDtypeStruct((N, D_OUT), jnp.float32),
    )(x, W, t2)


def _tc_combine_body(a_ref, b_ref, out_ref):
    out_ref[...] = a_ref[0] + b_ref[0]


def _tc_combine(halves):
    return pl.pallas_call(
        _tc_combine_body,
        grid=(N // BN,),
        in_specs=[
            pl.BlockSpec((1, BN, D_OUT), lambda i: (0, i, 0)),
            pl.BlockSpec((1, BN, D_OUT), lambda i: (1, i, 0)),
        ],
        out_specs=pl.BlockSpec((BN, D_OUT), lambda i: (i, 0)),
        out_shape=jax.ShapeDtypeStruct((N, D_OUT), jnp.float32),
    )(halves, halves)


def _scale_chunk(gbuf, sbuf, w_v, j):
    def group_body(g, carry2):
        wv = w_v[j, pl.ds(g * 16, 16)]
        for i in range(16):
            e = g * 16 + i
            ii = jnp.full((16,), i, jnp.int32)
            wb = lax.gather(
                wv, ii[:, None],
                lax.GatherDimensionNumbers(
                    offset_dims=(), collapsed_slice_dims=(0,),
                    start_index_map=(0,)),
                (1,),
                mode=lax.GatherScatterMode.PROMISE_IN_BOUNDS)
            for q in range(D_OUT // 16):
                sbuf[e, pl.ds(q * 16, 16)] = gbuf[e, pl.ds(q * 16, 16)] * wb
        return carry2

    lax.fori_loop(0, CHUNK // 16, group_body, 0)


def _sc_body(sup_ref, src_ref, dst_ref, w_ref, binit_ref, out_ref,
             acc, src_v, dst_v, w_v, gbuf,
             gsem0):
    c = lax.axis_index("c")
    tid = lax.axis_index("s")

    pltpu.sync_copy(binit_ref.at[c], acc.at[pl.ds(tid * ROWS_PER_TILE, ROWS_PER_TILE)])

    @pl.when(tid == NUM_TILES - 1)
    def _init_rem():
        pltpu.sync_copy(binit_ref.at[c, pl.ds(0, ROWS_REM)],
                        acc.at[pl.ds(NUM_TILES * ROWS_PER_TILE, ROWS_REM)])

    plsc.subcore_barrier()

    pltpu.sync_copy(src_ref.at[c, tid], src_v)
    pltpu.sync_copy(dst_ref.at[c, tid], dst_v)
    pltpu.sync_copy(w_ref.at[c, tid], w_v)

    def chunk_body(k, carry):
        pltpu.async_copy(sup_ref.at[src_v.at[k]], gbuf, gsem0).wait()
        _scale_chunk(gbuf, gbuf, w_v, k)
        pltpu.sync_copy(gbuf, acc.at[dst_v.at[k]], add=True)
        return carry

    lax.fori_loop(0, CHUNKS_PER_WORKER, chunk_body, 0)

    plsc.subcore_barrier()

    r0 = tid * ROWS_PER_TILE
    pltpu.sync_copy(
        acc.at[pl.ds(r0, ROWS_PER_TILE), :],
        out_ref.at[c, pl.ds(r0, ROWS_PER_TILE), :])

    @pl.when(tid == NUM_TILES - 1)
    def _out_rem():
        rr = NUM_TILES * ROWS_PER_TILE
        pltpu.sync_copy(acc.at[pl.ds(rr, ROWS_REM), :],
                        out_ref.at[c, pl.ds(rr, ROWS_REM), :])


def _sc_spmm(support, srcr, dstr, wr, binit):
    mesh = plsc.VectorSubcoreMesh(core_axis_name="c", subcore_axis_name="s")
    kern = pl.kernel(
        _sc_body,
        mesh=mesh,
        out_type=jax.ShapeDtypeStruct((2, N, D_OUT), jnp.float32),
        scratch_types=[
            pltpu.VMEM_SHARED((N, D_OUT), jnp.float32),
            pltpu.VMEM((CHUNKS_PER_WORKER, CHUNK), jnp.int32),
            pltpu.VMEM((CHUNKS_PER_WORKER, CHUNK), jnp.int32),
            pltpu.VMEM((CHUNKS_PER_WORKER, CHUNK), jnp.float32),
            pltpu.VMEM((CHUNK, D_OUT), jnp.float32),
            pltpu.SemaphoreType.DMA,
        ],
    )
    return kern(support, srcr, dstr, wr, binit)


def kernel(input, edge_index, edge_weight, t, W, b):
    x = input.astype(jnp.float32)
    t2 = t.reshape(N, 1)
    support = _tc_support(x, W, t2)

    src = edge_index[0].astype(jnp.int32).reshape(NUM_WORKERS, EDGES_PER_WORKER)
    dst = edge_index[1].astype(jnp.int32).reshape(NUM_WORKERS, EDGES_PER_WORKER)
    w = edge_weight.reshape(NUM_WORKERS, EDGES_PER_WORKER)
    pad = EDGES_PAD - EDGES_PER_WORKER
    eshape = (NUM_CORES, NUM_TILES, CHUNKS_PER_WORKER, CHUNK)
    srcr = jnp.pad(src, ((0, 0), (0, pad))).reshape(eshape)
    dstr = jnp.pad(dst, ((0, 0), (0, pad))).reshape(eshape)
    wr = jnp.pad(w, ((0, 0), (0, pad))).reshape(eshape)

    binit = jnp.stack([
        jnp.broadcast_to(b.reshape(1, D_OUT), (ROWS_PER_TILE, D_OUT)),
        jnp.zeros((ROWS_PER_TILE, D_OUT), jnp.float32),
    ])

    halves = _sc_spmm(support, srcr, dstr, wr, binit)
    return _tc_combine(halves)
